# baseline (speedup 1.0000x reference)
import functools

import jax
import jax.numpy as jnp
from jax.experimental import pallas as pl
from jax.experimental.pallas import tpu as pltpu

_BQ = 2048
_BK = 1024
_CK = 256
_BIG = 3.0e38


def _minacc_kernel(am_ref, bm_ref, out_ref, b2_ref):
    i = pl.program_id(0)
    j = pl.program_id(1)

    am = am_ref[...]
    bm = bm_ref[...]

    @pl.when(i == 0)
    def _():
        bf = bm.astype(jnp.float32)
        b2 = jnp.sum(bf * bf, axis=0, keepdims=True)
        b2_ref[:, pl.ds(j * _BK, _BK)] = b2.astype(jnp.bfloat16)

    @pl.when(j == 0)
    def _():
        out_ref[...] = jnp.full((_BQ, _CK), _BIG, jnp.bfloat16)

    ms = []
    for c in range(_BK // _CK):
        g = jnp.dot(
            am, bm[:, c * _CK : (c + 1) * _CK], preferred_element_type=jnp.float32
        )
        b2 = b2_ref[:, pl.ds(j * _BK + c * _CK, _CK)]
        ms.append(b2 + g.astype(jnp.bfloat16))
    while len(ms) > 1:
        ms = [jnp.minimum(ms[t], ms[t + 1]) for t in range(0, len(ms), 2)]
    out_ref[...] = jnp.minimum(out_ref[...], ms[0])


def _mean_kernel(acc_ref, am_ref, out_ref, *, inv_q):
    i = pl.program_id(0)
    nq = pl.num_programs(0)
    af = am_ref[...].astype(jnp.float32)
    a2 = 0.25 * jnp.sum(af * af, axis=1, keepdims=True)
    mn = jnp.min(acc_ref[...], axis=1, keepdims=True).astype(jnp.float32)
    d2 = a2 + mn
    psum = jnp.sum(jnp.sqrt(jnp.maximum(d2, 1e-12))).reshape(1, 1)
    tot = jnp.where(i == 0, psum, out_ref[...] + psum)
    out_ref[...] = jnp.where(i == nq - 1, tot * inv_q, tot)


@jax.jit
def kernel(target_embeddings, target_slice_idx, cluster_centers):
    del target_slice_idx
    q, d = target_embeddings.shape
    k = cluster_centers.shape[0]
    a_mm = (-2.0 * target_embeddings).astype(jnp.bfloat16)
    b_mm = cluster_centers.T.astype(jnp.bfloat16)

    minacc = pl.pallas_call(
        _minacc_kernel,
        grid=(q // _BQ, k // _BK),
        in_specs=[
            pl.BlockSpec((_BQ, d), lambda i, j: (i, 0)),
            pl.BlockSpec((d, _BK), lambda i, j: (0, j)),
        ],
        out_specs=pl.BlockSpec((_BQ, _CK), lambda i, j: (i, 0)),
        out_shape=jax.ShapeDtypeStruct((q, _CK), jnp.bfloat16),
        scratch_shapes=[pltpu.VMEM((1, k), jnp.bfloat16)],
    )(a_mm, b_mm)

    out = pl.pallas_call(
        functools.partial(_mean_kernel, inv_q=1.0 / q),
        grid=(q // _BQ,),
        in_specs=[
            pl.BlockSpec((_BQ, _CK), lambda i: (i, 0)),
            pl.BlockSpec((_BQ, d), lambda i: (i, 0)),
        ],
        out_specs=pl.BlockSpec((1, 1), lambda i: (0, 0)),
        out_shape=jax.ShapeDtypeStruct((1, 1), jnp.float32),
    )(minacc, a_mm)
    return out[0, 0]


# speedup vs baseline: 1.1837x; 1.1837x over previous
import functools

import jax
import jax.numpy as jnp
from jax.experimental import pallas as pl
from jax.experimental.pallas import tpu as pltpu

_BQ = 2048
_BK = 1024
_CK = 256
_BB = 2048
_BIG = 3.0e38
_F8 = jnp.float8_e4m3fn


def _b2_kernel(bm_ref, out_ref):
    bf = bm_ref[...].astype(jnp.float32)
    b2 = jnp.sum(bf * bf, axis=0, keepdims=True)
    out_ref[...] = b2.astype(jnp.bfloat16)


def _nn_loss_kernel(am_ref, bm_ref, b2_ref, out_ref, acc_ref, *, inv_q):
    i = pl.program_id(0)
    j = pl.program_id(1)
    nq = pl.num_programs(0)
    nk = pl.num_programs(1)

    am = am_ref[...]
    bm = bm_ref[...]

    @pl.when(j == 0)
    def _():
        acc_ref[...] = jnp.full((_BQ, _CK), _BIG, jnp.bfloat16)

    ms = []
    for c in range(_BK // _CK):
        g = jnp.dot(
            am, bm[:, c * _CK : (c + 1) * _CK], preferred_element_type=jnp.float32
        )
        b2 = b2_ref[:, pl.ds(j * _BK + c * _CK, _CK)]
        ms.append(b2 + g.astype(jnp.bfloat16))
    while len(ms) > 1:
        ms = [jnp.minimum(ms[t], ms[t + 1]) for t in range(0, len(ms), 2)]
    acc_ref[...] = jnp.minimum(acc_ref[...], ms[0])

    @pl.when(j == nk - 1)
    def _():
        af = am.astype(jnp.float32)
        a2 = 0.25 * jnp.sum(af * af, axis=1, keepdims=True)
        mn = jnp.min(acc_ref[...], axis=1, keepdims=True).astype(jnp.float32)
        d2 = a2 + mn
        psum = jnp.sum(jnp.sqrt(jnp.maximum(d2, 1e-12))).reshape(1, 1)
        tot = jnp.where(i == 0, psum, out_ref[...] + psum)
        out_ref[...] = jnp.where(i == nq - 1, tot * inv_q, tot)


@jax.jit
def kernel(target_embeddings, target_slice_idx, cluster_centers):
    del target_slice_idx
    q, d = target_embeddings.shape
    k = cluster_centers.shape[0]
    a_f8 = (-2.0 * target_embeddings).astype(_F8)
    b_t = cluster_centers.T
    b_f8 = b_t.astype(_F8)
    b_bf = b_t.astype(jnp.bfloat16)

    b2row = pl.pallas_call(
        _b2_kernel,
        grid=(k // _BB,),
        in_specs=[pl.BlockSpec((d, _BB), lambda i: (0, i))],
        out_specs=pl.BlockSpec((1, _BB), lambda i: (0, i)),
        out_shape=jax.ShapeDtypeStruct((1, k), jnp.bfloat16),
    )(b_bf)

    out = pl.pallas_call(
        functools.partial(_nn_loss_kernel, inv_q=1.0 / q),
        grid=(q // _BQ, k // _BK),
        in_specs=[
            pl.BlockSpec((_BQ, d), lambda i, j: (i, 0)),
            pl.BlockSpec((d, _BK), lambda i, j: (0, j)),
            pl.BlockSpec((1, k), lambda i, j: (0, 0)),
        ],
        out_specs=pl.BlockSpec((1, 1), lambda i, j: (0, 0)),
        out_shape=jax.ShapeDtypeStruct((1, 1), jnp.float32),
        scratch_shapes=[pltpu.VMEM((_BQ, _CK), jnp.bfloat16)],
    )(a_f8, b_f8, b2row)
    return out[0, 0]


# speedup vs baseline: 1.5403x; 1.3013x over previous
import functools

import jax
import jax.numpy as jnp
from jax.experimental import pallas as pl

_BQ = 2048
_CK = 512
_BB = 2048
_F8 = jnp.float8_e4m3fn


def _b2_kernel(bm_ref, out_ref):
    bf = bm_ref[...].astype(jnp.float32)
    b2 = jnp.sum(bf * bf, axis=0, keepdims=True)
    out_ref[...] = b2.astype(jnp.bfloat16)


def _nn_loss_kernel(am_ref, bm_ref, b2_ref, out_ref, *, k, inv_q):
    i = pl.program_id(0)
    nq = pl.num_programs(0)

    am = am_ref[...]
    bm = bm_ref[...]

    m = None
    for c in range(k // _CK):
        g = jnp.dot(
            am, bm[:, c * _CK : (c + 1) * _CK], preferred_element_type=jnp.float32
        )
        b2 = b2_ref[:, c * _CK : (c + 1) * _CK]
        t = b2 + g.astype(jnp.bfloat16)
        m = t if m is None else jnp.minimum(m, t)

    af = am.astype(jnp.float32)
    a2 = 0.25 * jnp.sum(af * af, axis=1, keepdims=True)
    mn = jnp.min(m, axis=1, keepdims=True).astype(jnp.float32)
    d2 = a2 + mn
    psum = jnp.sum(jnp.sqrt(jnp.maximum(d2, 1e-12))).reshape(1, 1)
    tot = jnp.where(i == 0, psum, out_ref[...] + psum)
    out_ref[...] = jnp.where(i == nq - 1, tot * inv_q, tot)


@jax.jit
def kernel(target_embeddings, target_slice_idx, cluster_centers):
    del target_slice_idx
    q, d = target_embeddings.shape
    k = cluster_centers.shape[0]
    a_f8 = (-2.0 * target_embeddings).astype(_F8)
    b_t = cluster_centers.T
    b_f8 = b_t.astype(_F8)
    b_bf = b_t.astype(jnp.bfloat16)

    b2row = pl.pallas_call(
        _b2_kernel,
        grid=(k // _BB,),
        in_specs=[pl.BlockSpec((d, _BB), lambda i: (0, i))],
        out_specs=pl.BlockSpec((1, _BB), lambda i: (0, i)),
        out_shape=jax.ShapeDtypeStruct((1, k), jnp.bfloat16),
    )(b_bf)

    out = pl.pallas_call(
        functools.partial(_nn_loss_kernel, k=k, inv_q=1.0 / q),
        grid=(q // _BQ,),
        in_specs=[
            pl.BlockSpec((_BQ, d), lambda i: (i, 0)),
            pl.BlockSpec((d, k), lambda i: (0, 0)),
            pl.BlockSpec((1, k), lambda i: (0, 0)),
        ],
        out_specs=pl.BlockSpec((1, 1), lambda i: (0, 0)),
        out_shape=jax.ShapeDtypeStruct((1, 1), jnp.float32),
    )(a_f8, b_f8, b2row)
    return out[0, 0]


# speedup vs baseline: 2.6751x; 1.7367x over previous
import functools

import jax
import jax.numpy as jnp
from jax.experimental import pallas as pl
from jax.experimental.pallas import tpu as pltpu

_BQ = 2048
_CK = 512
_F8 = jnp.float8_e4m3fn


def _nn_loss_kernel(a_ref, b_ref, out_ref, bt_ref, b2_ref, *, k, inv_q):
    i = pl.program_id(0)
    nq = pl.num_programs(0)

    @pl.when(i == 0)
    def _():
        bt = b_ref[...].T
        bt_ref[...] = (-2.0 * bt).astype(_F8)
        b2_ref[...] = jnp.sum(bt * bt, axis=0, keepdims=True).astype(jnp.bfloat16)

    am = a_ref[...]
    af8 = am.astype(_F8)

    m = None
    for c in range(k // _CK):
        g = jnp.dot(
            af8,
            bt_ref[:, c * _CK : (c + 1) * _CK],
            preferred_element_type=jnp.float32,
        )
        b2 = b2_ref[:, c * _CK : (c + 1) * _CK]
        t = b2 + g.astype(jnp.bfloat16)
        m = t if m is None else jnp.minimum(m, t)

    a2 = jnp.sum(am * am, axis=1, keepdims=True)
    d2 = a2 + jnp.min(m, axis=1, keepdims=True).astype(jnp.float32)
    psum = jnp.sum(jnp.sqrt(jnp.maximum(d2, 1e-12))).reshape(1, 1)
    tot = jnp.where(i == 0, psum, out_ref[...] + psum)
    out_ref[...] = jnp.where(i == nq - 1, tot * inv_q, tot)


@jax.jit
def kernel(target_embeddings, target_slice_idx, cluster_centers):
    del target_slice_idx
    q, d = target_embeddings.shape
    k = cluster_centers.shape[0]

    out = pl.pallas_call(
        functools.partial(_nn_loss_kernel, k=k, inv_q=1.0 / q),
        grid=(q // _BQ,),
        in_specs=[
            pl.BlockSpec((_BQ, d), lambda i: (i, 0)),
            pl.BlockSpec((k, d), lambda i: (0, 0)),
        ],
        out_specs=pl.BlockSpec((1, 1), lambda i: (0, 0)),
        out_shape=jax.ShapeDtypeStruct((1, 1), jnp.float32),
        scratch_shapes=[
            pltpu.VMEM((d, k), _F8),
            pltpu.VMEM((1, k), jnp.bfloat16),
        ],
    )(target_embeddings, cluster_centers)
    return out[0, 0]


# speedup vs baseline: 2.7368x; 1.0231x over previous
import functools

import jax
import jax.numpy as jnp
from jax.experimental import pallas as pl
from jax.experimental.pallas import tpu as pltpu

_BQ = 8192
_CK = 256
_F8 = jnp.float8_e4m3fn


def _nn_loss_kernel(a_ref, b_ref, out_ref, bt_ref, b2_ref, *, k, inv_q):
    i = pl.program_id(0)
    nq = pl.num_programs(0)

    @pl.when(i == 0)
    def _():
        bt = b_ref[...].T
        bt_ref[...] = (-2.0 * bt).astype(_F8)
        b2_ref[...] = jnp.sum(bt * bt, axis=0, keepdims=True).astype(jnp.bfloat16)

    am = a_ref[...]
    af8 = am.astype(_F8)

    m = None
    for c in range(k // _CK):
        g = jnp.dot(
            af8,
            bt_ref[:, c * _CK : (c + 1) * _CK],
            preferred_element_type=jnp.float32,
        )
        b2 = b2_ref[:, c * _CK : (c + 1) * _CK]
        t = b2 + g.astype(jnp.bfloat16)
        m = t if m is None else jnp.minimum(m, t)

    a2 = jnp.sum(am * am, axis=1, keepdims=True)
    d2 = a2 + jnp.min(m, axis=1, keepdims=True).astype(jnp.float32)
    psum = jnp.sum(jnp.sqrt(jnp.maximum(d2, 1e-12))).reshape(1, 1)
    tot = jnp.where(i == 0, psum, out_ref[...] + psum)
    out_ref[...] = jnp.where(i == nq - 1, tot * inv_q, tot)


@jax.jit
def kernel(target_embeddings, target_slice_idx, cluster_centers):
    del target_slice_idx
    q, d = target_embeddings.shape
    k = cluster_centers.shape[0]

    out = pl.pallas_call(
        functools.partial(_nn_loss_kernel, k=k, inv_q=1.0 / q),
        grid=(q // _BQ,),
        in_specs=[
            pl.BlockSpec((_BQ, d), lambda i: (i, 0)),
            pl.BlockSpec((k, d), lambda i: (0, 0)),
        ],
        out_specs=pl.BlockSpec((1, 1), lambda i: (0, 0)),
        out_shape=jax.ShapeDtypeStruct((1, 1), jnp.float32),
        scratch_shapes=[
            pltpu.VMEM((d, k), _F8),
            pltpu.VMEM((1, k), jnp.bfloat16),
        ],
    )(target_embeddings, cluster_centers)
    return out[0, 0]
